# Initial kernel scaffold; baseline (speedup 1.0000x reference)
import functools

import jax
import jax.numpy as jnp
from jax import lax
from jax.experimental import pallas as pl
from jax.experimental.pallas import tpu as pltpu
from jax.experimental.pallas import tpu_sc as plsc

N_NODES = 10000
N_EDGES = 320000
D = 128
L = 16
NC = 2
NS = 16
EDGES_PER_W = N_EDGES // (NC * NS)
CHUNK = 80
NCHUNKS = EDGES_PER_W // CHUNK
ROWS_PER_TILE = N_NODES // NS
ZROWS = 125

_mesh = plsc.VectorSubcoreMesh(core_axis_name="c", subcore_axis_name="s")


@functools.partial(
    pl.kernel,
    out_type=jax.ShapeDtypeStruct((NC, N_NODES, D), jnp.float32),
    mesh=_mesh,
    scratch_types=[
        pltpu.VMEM((CHUNK,), jnp.int32),
        pltpu.VMEM((CHUNK,), jnp.int32),
        pltpu.SMEM((CHUNK,), jnp.float32),
        pltpu.VMEM((CHUNK, D), jnp.float32),
        pltpu.VMEM((ZROWS, D), jnp.float32),
        pltpu.VMEM_SHARED((N_NODES, D), jnp.float32),
        pltpu.SemaphoreType.DMA,
    ],
)
def _propagate_sc(x_hbm, src_hbm, dst_hbm, w_hbm, out_hbm,
                  srcv, dstv, wsm, rows, stage, acc, sem):
    c = lax.axis_index("c")
    s = lax.axis_index("s")

    zero = jnp.zeros((L,), jnp.float32)

    def _zrow(i, _):
        for j in range(D // L):
            stage[i, pl.ds(j * L, L)] = zero
        return 0

    lax.fori_loop(0, ZROWS, _zrow, 0)
    for k in range(ROWS_PER_TILE // ZROWS):
        pltpu.sync_copy(stage, acc.at[pl.ds(s * ROWS_PER_TILE + k * ZROWS, ZROWS)])
    plsc.subcore_barrier()

    base0 = c * (N_EDGES // NC) + s * EDGES_PER_W

    def _chunk(i, _):
        base = base0 + i * CHUNK
        pltpu.sync_copy(src_hbm.at[pl.ds(base, CHUNK)], srcv)
        pltpu.sync_copy(dst_hbm.at[pl.ds(base, CHUNK)], dstv)
        pltpu.sync_copy(w_hbm.at[pl.ds(base, CHUNK)], wsm)
        pltpu.async_copy(x_hbm.at[srcv], rows, sem).wait()

        def _edge(e, _):
            wv = wsm[e]
            for j in range(D // L):
                sl = pl.ds(j * L, L)
                rows[e, sl] = rows[e, sl] * wv
            return 0

        lax.fori_loop(0, CHUNK, _edge, 0)
        pltpu.sync_copy(rows, acc.at[dstv], add=True)
        return 0

    lax.fori_loop(0, NCHUNKS, _chunk, 0)
    plsc.subcore_barrier()

    for k in range(ROWS_PER_TILE // ZROWS):
        r0 = s * ROWS_PER_TILE + k * ZROWS
        pltpu.sync_copy(acc.at[pl.ds(r0, ZROWS)], stage)
        pltpu.sync_copy(stage, out_hbm.at[c, pl.ds(r0, ZROWS)])


def _dense_relu_body(p_ref, wt_ref, b_ref, out_ref):
    h = p_ref[0] + p_ref[1]
    y = jnp.dot(h, wt_ref[...], preferred_element_type=jnp.float32) + b_ref[...]
    out_ref[...] = jnp.maximum(y, 0.0)


def _dense_final_body(p_ref, wt_ref, b_ref, out_ref):
    h = p_ref[0] + p_ref[1]
    y = jnp.dot(h, wt_ref[...], preferred_element_type=jnp.float32) + b_ref[...]
    m = jnp.max(y, axis=1, keepdims=True)
    e = y - m
    lse = jnp.log(jnp.sum(jnp.exp(e), axis=1, keepdims=True))
    out_ref[...] = e - lse


_ROW_BLK = 2500


def _dense_call(body, p, wt, b):
    grid = (N_NODES // _ROW_BLK,)
    return pl.pallas_call(
        body,
        grid=grid,
        in_specs=[
            pl.BlockSpec((NC, _ROW_BLK, D), lambda i: (0, i, 0)),
            pl.BlockSpec((D, D), lambda i: (0, 0)),
            pl.BlockSpec((1, D), lambda i: (0, 0)),
        ],
        out_specs=pl.BlockSpec((_ROW_BLK, D), lambda i: (i, 0)),
        out_shape=jax.ShapeDtypeStruct((N_NODES, D), jnp.float32),
    )(p, wt, b)


def kernel(x, edge_index, edge_weight, W1, b1, W2, b2, W3, b3):
    src = edge_index[0].astype(jnp.int32)
    dst = edge_index[1].astype(jnp.int32)
    w = edge_weight.astype(jnp.float32)

    p1 = _propagate_sc(x, src, dst, w)
    h1 = _dense_call(_dense_relu_body, p1, W1.T, b1.reshape(1, D))
    p2 = _propagate_sc(h1, src, dst, w)
    h2 = _dense_call(_dense_relu_body, p2, W2.T, b2.reshape(1, D))
    p3 = _propagate_sc(h2, src, dst, w)
    return _dense_call(_dense_final_body, p3, W3.T, b3.reshape(1, D))


# speedup vs baseline: 1.7734x; 1.7734x over previous
import functools

import jax
import jax.numpy as jnp
from jax import lax
from jax.experimental import pallas as pl
from jax.experimental.pallas import tpu as pltpu
from jax.experimental.pallas import tpu_sc as plsc

N_NODES = 10000
N_EDGES = 320000
D = 128
L = 16
NC = 2
NS = 16
NHALF = N_NODES // NC
ACC_ROWS = NHALF + 8
EDGES_PER_TILE = N_EDGES // NS
CHUNK = 80
NCHUNKS = EDGES_PER_TILE // CHUNK
ZROWS = 312
TAIL = NHALF - NS * ZROWS

_mesh = plsc.VectorSubcoreMesh(core_axis_name="c", subcore_axis_name="s")


@functools.partial(
    pl.kernel,
    out_type=jax.ShapeDtypeStruct((NC, NHALF, D), jnp.float32),
    mesh=_mesh,
    scratch_types=[
        pltpu.VMEM((CHUNK,), jnp.int32),
        pltpu.VMEM((CHUNK,), jnp.int32),
        pltpu.VMEM((CHUNK,), jnp.float32),
        pltpu.VMEM((CHUNK, D), jnp.float32),
        pltpu.VMEM((ZROWS, D), jnp.float32),
        pltpu.VMEM((TAIL, D), jnp.float32),
        pltpu.VMEM_SHARED((ACC_ROWS, D), jnp.float32),
        pltpu.SemaphoreType.DMA,
    ],
    compiler_params=pltpu.CompilerParams(needs_layout_passes=False),
)
def _propagate_sc(x_hbm, src_hbm, dst_hbm, w_hbm, out_hbm,
                  srcv, dstv, wv, rows, stage, stail, acc, sem):
    c = lax.axis_index("c")
    s = lax.axis_index("s")

    zero = jnp.zeros((L,), jnp.float32)

    def _zrow(i, _):
        for j in range(D // L):
            stage[i, pl.ds(j * L, L)] = zero
        return 0

    lax.fori_loop(0, ZROWS, _zrow, 0)
    for i in range(TAIL):
        for j in range(D // L):
            stail[i, pl.ds(j * L, L)] = zero
    pltpu.sync_copy(stage, acc.at[pl.ds(s * ZROWS, ZROWS)])

    @pl.when(s == NS - 1)
    def _():
        pltpu.sync_copy(stail, acc.at[pl.ds(NS * ZROWS, TAIL)])
        pltpu.sync_copy(stail, acc.at[pl.ds(NHALF, ACC_ROWS - NHALF)])

    plsc.subcore_barrier()

    base0 = s * EDGES_PER_TILE
    lo = c * NHALF

    def _chunk(i, _):
        base = base0 + i * CHUNK
        pltpu.sync_copy(src_hbm.at[pl.ds(base, CHUNK)], srcv)
        pltpu.sync_copy(dst_hbm.at[pl.ds(base, CHUNK)], dstv)
        pltpu.sync_copy(w_hbm.at[pl.ds(base, CHUNK)], wv)
        for k in range(CHUNK // L):
            sl = pl.ds(k * L, L)
            local = dstv[sl] - lo
            inb = (local >= 0) & (local < NHALF)
            dstv[sl] = jnp.where(inb, local, NHALF)
        pltpu.async_copy(x_hbm.at[srcv], rows, sem).wait()

        def _edge(e, _):
            wb = plsc.load_gather(wv, [jnp.full((L,), e, jnp.int32)])
            for j in range(D // L):
                sl = pl.ds(j * L, L)
                rows[e, sl] = rows[e, sl] * wb
            return 0

        lax.fori_loop(0, CHUNK, _edge, 0)
        pltpu.sync_copy(rows, acc.at[dstv], add=True)
        return 0

    lax.fori_loop(0, NCHUNKS, _chunk, 0)
    plsc.subcore_barrier()

    r0 = s * ZROWS
    pltpu.sync_copy(acc.at[pl.ds(r0, ZROWS)], stage)
    pltpu.sync_copy(stage, out_hbm.at[c, pl.ds(r0, ZROWS)])

    @pl.when(s == NS - 1)
    def _():
        pltpu.sync_copy(acc.at[pl.ds(NS * ZROWS, TAIL)], stail)
        pltpu.sync_copy(stail, out_hbm.at[c, pl.ds(NS * ZROWS, TAIL)])


def _dense_relu_body(p_ref, wt_ref, b_ref, out_ref):
    y = jnp.dot(p_ref[...], wt_ref[...],
                preferred_element_type=jnp.float32) + b_ref[...]
    out_ref[...] = jnp.maximum(y, 0.0)


def _dense_final_body(p_ref, wt_ref, b_ref, out_ref):
    y = jnp.dot(p_ref[...], wt_ref[...],
                preferred_element_type=jnp.float32) + b_ref[...]
    m = jnp.max(y, axis=1, keepdims=True)
    e = y - m
    lse = jnp.log(jnp.sum(jnp.exp(e), axis=1, keepdims=True))
    out_ref[...] = e - lse


_ROW_BLK = 2000


def _dense_call(body, p, wt, b):
    grid = (N_NODES // _ROW_BLK,)
    return pl.pallas_call(
        body,
        grid=grid,
        in_specs=[
            pl.BlockSpec((_ROW_BLK, D), lambda i: (i, 0)),
            pl.BlockSpec((D, D), lambda i: (0, 0)),
            pl.BlockSpec((1, D), lambda i: (0, 0)),
        ],
        out_specs=pl.BlockSpec((_ROW_BLK, D), lambda i: (i, 0)),
        out_shape=jax.ShapeDtypeStruct((N_NODES, D), jnp.float32),
    )(p, wt, b)


def kernel(x, edge_index, edge_weight, W1, b1, W2, b2, W3, b3):
    src = edge_index[0].astype(jnp.int32)
    dst = edge_index[1].astype(jnp.int32)
    w = edge_weight.astype(jnp.float32)

    p1 = _propagate_sc(x, src, dst, w).reshape(N_NODES, D)
    h1 = _dense_call(_dense_relu_body, p1, W1.T, b1.reshape(1, D))
    p2 = _propagate_sc(h1, src, dst, w).reshape(N_NODES, D)
    h2 = _dense_call(_dense_relu_body, p2, W2.T, b2.reshape(1, D))
    p3 = _propagate_sc(h2, src, dst, w).reshape(N_NODES, D)
    return _dense_call(_dense_final_body, p3, W3.T, b3.reshape(1, D))


# speedup vs baseline: 4.7259x; 2.6649x over previous
import functools

import jax
import jax.numpy as jnp
from jax import lax
from jax.experimental import pallas as pl
from jax.experimental.pallas import tpu as pltpu
from jax.experimental.pallas import tpu_sc as plsc

N_NODES = 10000
N_EDGES = 320000
D = 128
L = 16
NC = 2
NS = 16
NHALF = N_NODES // NC
ACC_ROWS = NHALF + 8
EDGES_PER_TILE = N_EDGES // NS
CHUNK = 80
NCHUNKS = EDGES_PER_TILE // CHUNK
ZROWS = 312
TAIL = NHALF - NS * ZROWS

_mesh = plsc.VectorSubcoreMesh(core_axis_name="c", subcore_axis_name="s")


@functools.partial(
    pl.kernel,
    out_type=jax.ShapeDtypeStruct((NC, NHALF, D), jnp.float32),
    mesh=_mesh,
    scratch_types=[
        pltpu.VMEM((CHUNK,), jnp.int32),
        pltpu.VMEM((CHUNK,), jnp.int32),
        pltpu.VMEM((CHUNK,), jnp.int32),
        pltpu.VMEM((CHUNK,), jnp.int32),
        pltpu.VMEM((CHUNK,), jnp.float32),
        pltpu.VMEM((CHUNK,), jnp.float32),
        pltpu.VMEM((CHUNK,), jnp.int32),
        pltpu.VMEM((CHUNK,), jnp.int32),
        pltpu.VMEM((CHUNK,), jnp.int32),
        pltpu.VMEM((CHUNK,), jnp.int32),
        pltpu.VMEM((CHUNK,), jnp.float32),
        pltpu.VMEM((CHUNK,), jnp.float32),
        pltpu.VMEM((CHUNK, D), jnp.float32),
        pltpu.VMEM((CHUNK, D), jnp.float32),
        pltpu.VMEM((ZROWS, D), jnp.float32),
        pltpu.VMEM((TAIL, D), jnp.float32),
        pltpu.VMEM_SHARED((ACC_ROWS, D), jnp.float32),
        pltpu.SemaphoreType.DMA,
        pltpu.SemaphoreType.DMA,
        pltpu.SemaphoreType.DMA,
        pltpu.SemaphoreType.DMA,
        pltpu.SemaphoreType.DMA,
        pltpu.SemaphoreType.DMA,
    ],
    compiler_params=pltpu.CompilerParams(needs_layout_passes=False),
)
def _propagate_sc(x_hbm, src_hbm, dst_hbm, w_hbm, out_hbm,
                  rsrc0, rsrc1, rdst0, rdst1, rw0, rw1,
                  srcv0, srcv1, dstv0, dstv1, wv0, wv1,
                  rows0, rows1, stage, stail, acc,
                  esem0, esem1, gsem0, gsem1, ssem0, ssem1):
    rsrc = (rsrc0, rsrc1)
    rdst = (rdst0, rdst1)
    rw = (rw0, rw1)
    srcv = (srcv0, srcv1)
    dstv = (dstv0, dstv1)
    wv = (wv0, wv1)
    rows = (rows0, rows1)
    esem = (esem0, esem1)
    gsem = (gsem0, gsem1)
    ssem = (ssem0, ssem1)
    c = lax.axis_index("c")
    s = lax.axis_index("s")

    zero = jnp.zeros((L,), jnp.float32)

    def _zrow(i, _):
        for j in range(D // L):
            stage[i, pl.ds(j * L, L)] = zero
        return 0

    lax.fori_loop(0, ZROWS, _zrow, 0)
    for i in range(TAIL):
        for j in range(D // L):
            stail[i, pl.ds(j * L, L)] = zero
    pltpu.sync_copy(stage, acc.at[pl.ds(s * ZROWS, ZROWS)])

    @pl.when(s == NS - 1)
    def _():
        pltpu.sync_copy(stail, acc.at[pl.ds(NS * ZROWS, TAIL)])
        pltpu.sync_copy(stail, acc.at[pl.ds(NHALF, ACC_ROWS - NHALF)])

    plsc.subcore_barrier()

    base0 = s * EDGES_PER_TILE
    lo = c * NHALF

    def _issue_e(ck, b):
        base = base0 + ck * CHUNK
        pltpu.async_copy(src_hbm.at[pl.ds(base, CHUNK)], rsrc[b], esem[b])
        pltpu.async_copy(dst_hbm.at[pl.ds(base, CHUNK)], rdst[b], esem[b])
        pltpu.async_copy(w_hbm.at[pl.ds(base, CHUNK)], rw[b], esem[b])

    def _wait_e(b):
        pltpu.make_async_copy(src_hbm.at[pl.ds(0, CHUNK)], rsrc[b],
                              esem[b]).wait()
        pltpu.make_async_copy(dst_hbm.at[pl.ds(0, CHUNK)], rdst[b],
                              esem[b]).wait()
        pltpu.make_async_copy(w_hbm.at[pl.ds(0, CHUNK)], rw[b],
                              esem[b]).wait()

    def _build(b):
        for k in range(CHUNK // L):
            sl = pl.ds(k * L, L)
            srcv[b][sl] = rsrc[b][sl]
            local = rdst[b][sl] - lo
            inb = (local >= 0) & (local < NHALF)
            dstv[b][sl] = jnp.where(inb, local, NHALF)
            wv[b][sl] = rw[b][sl]

    def _issue_g(b):
        pltpu.async_copy(x_hbm.at[srcv[b]], rows[b], gsem[b])

    def _wait_g(b):
        pltpu.make_async_copy(x_hbm.at[pl.ds(0, CHUNK)], rows[b],
                              gsem[b]).wait()

    def _scale(b):
        def _edge(e, _):
            wb = plsc.load_gather(wv[b], [jnp.full((L,), e, jnp.int32)])
            for j in range(D // L):
                sl = pl.ds(j * L, L)
                rows[b][e, sl] = rows[b][e, sl] * wb
            return 0

        lax.fori_loop(0, CHUNK, _edge, 0)

    def _issue_a(b):
        pltpu.async_copy(rows[b], acc.at[dstv[b]], ssem[b], add=True)

    def _wait_a(b):
        pltpu.make_async_copy(x_hbm.at[pl.ds(0, CHUNK)], rows[b],
                              ssem[b]).wait()

    _issue_e(0, 0)
    _wait_e(0)
    _build(0)
    _issue_g(0)
    _issue_e(1, 1)

    def _pair(i, _):
        for b in range(2):
            p, q = b, 1 - b
            ck = 2 * i + b

            @pl.when(ck + 1 < NCHUNKS)
            def _():
                _wait_e(q)
                if b == 1:
                    _wait_a(q)
                else:
                    @pl.when(ck >= 1)
                    def _():
                        _wait_a(q)
                _build(q)
                _issue_g(q)

                @pl.when(ck + 2 < NCHUNKS)
                def _():
                    _issue_e(ck + 2, p)

            _wait_g(p)
            _scale(p)
            _issue_a(p)
        return 0

    lax.fori_loop(0, NCHUNKS // 2, _pair, 0)
    _wait_a(0)
    _wait_a(1)
    plsc.subcore_barrier()

    r0 = s * ZROWS
    pltpu.sync_copy(acc.at[pl.ds(r0, ZROWS)], stage)
    pltpu.sync_copy(stage, out_hbm.at[c, pl.ds(r0, ZROWS)])

    @pl.when(s == NS - 1)
    def _():
        pltpu.sync_copy(acc.at[pl.ds(NS * ZROWS, TAIL)], stail)
        pltpu.sync_copy(stail, out_hbm.at[c, pl.ds(NS * ZROWS, TAIL)])


def _dense_relu_body(p_ref, wt_ref, b_ref, out_ref):
    y = jnp.dot(p_ref[...], wt_ref[...],
                preferred_element_type=jnp.float32) + b_ref[...]
    out_ref[...] = jnp.maximum(y, 0.0)


def _dense_final_body(p_ref, wt_ref, b_ref, out_ref):
    y = jnp.dot(p_ref[...], wt_ref[...],
                preferred_element_type=jnp.float32) + b_ref[...]
    m = jnp.max(y, axis=1, keepdims=True)
    e = y - m
    lse = jnp.log(jnp.sum(jnp.exp(e), axis=1, keepdims=True))
    out_ref[...] = e - lse


_ROW_BLK = 2000


def _dense_call(body, p, wt, b):
    grid = (N_NODES // _ROW_BLK,)
    return pl.pallas_call(
        body,
        grid=grid,
        in_specs=[
            pl.BlockSpec((_ROW_BLK, D), lambda i: (i, 0)),
            pl.BlockSpec((D, D), lambda i: (0, 0)),
            pl.BlockSpec((1, D), lambda i: (0, 0)),
        ],
        out_specs=pl.BlockSpec((_ROW_BLK, D), lambda i: (i, 0)),
        out_shape=jax.ShapeDtypeStruct((N_NODES, D), jnp.float32),
    )(p, wt, b)


def kernel(x, edge_index, edge_weight, W1, b1, W2, b2, W3, b3):
    src = edge_index[0].astype(jnp.int32)
    dst = edge_index[1].astype(jnp.int32)
    w = edge_weight.astype(jnp.float32)

    p1 = _propagate_sc(x, src, dst, w).reshape(N_NODES, D)
    h1 = _dense_call(_dense_relu_body, p1, W1.T, b1.reshape(1, D))
    p2 = _propagate_sc(h1, src, dst, w).reshape(N_NODES, D)
    h2 = _dense_call(_dense_relu_body, p2, W2.T, b2.reshape(1, D))
    p3 = _propagate_sc(h2, src, dst, w).reshape(N_NODES, D)
    return _dense_call(_dense_final_body, p3, W3.T, b3.reshape(1, D))
